# baseline (speedup 1.0000x reference)
import functools

import jax
import jax.numpy as jnp
from jax import lax
from jax.experimental import pallas as pl
from jax.experimental.pallas import tpu as pltpu
from jax.experimental.pallas import tpu_sc as plsc


_SQRT_HALF = 0.7071067811865476


def _gelu(v):
    return 0.5 * v * (1.0 + lax.erf(v * _SQRT_HALF))


def _sc_spmm(nf, row, col, ev):
    n, d = nf.shape
    e = row.shape[0]
    nc, ns = 2, 16
    nw = nc * ns
    chunk = 128
    ept = e // nw
    ncp = -(-ept // chunk)
    eptp = ncp * chunk
    rpt = n // ns
    zr = 25
    nz = rpt // zr

    def ptile(v):
        return jnp.pad(v.reshape(nw, ept), ((0, 0), (0, eptp - ept))
                       ).reshape(nw, ncp, chunk)

    pk = jnp.zeros((nw, ncp, 8, chunk), jnp.int32)
    pk = pk.at[:, :, 0].set(ptile(col))
    pk = pk.at[:, :, 1].set(ptile(row))
    pk = pk.at[:, :, 2].set(ptile(lax.bitcast_convert_type(ev, jnp.int32)))

    mesh = plsc.VectorSubcoreMesh(core_axis_name="c", subcore_axis_name="s")

    @functools.partial(
        pl.kernel,
        mesh=mesh,
        out_type=jax.ShapeDtypeStruct((nc, n, d), jnp.float32),
        scratch_types=[
            pltpu.VMEM((8, chunk), jnp.int32),
            pltpu.VMEM((chunk,), jnp.int32),
            pltpu.VMEM((chunk,), jnp.int32),
            pltpu.VMEM((chunk, d), jnp.float32),
            pltpu.VMEM((zr, d), jnp.float32),
            pltpu.VMEM_SHARED((n, d), jnp.float32),
            pltpu.SemaphoreType.DMA,
        ],
    )
    def spmm(nf_hbm, pk_hbm, out_hbm,
             pkv, colv, rowv, rowsv, zbuf, acc, sem):
        cid = lax.axis_index("c")
        sid = lax.axis_index("s")
        tid = cid * ns + sid

        z16 = jnp.zeros((16,), jnp.float32)
        for i in range(zr):
            for j in range(d // 16):
                zbuf[i, pl.ds(j * 16, 16)] = z16
        zbase = sid * rpt

        def zero_body(i, carry):
            pltpu.sync_copy(zbuf, acc.at[pl.ds(zbase + i * zr, zr)])
            return carry

        lax.fori_loop(0, nz, zero_body, 0)
        plsc.subcore_barrier()

        def chunk_body(i, carry):
            pltpu.sync_copy(pk_hbm.at[tid, i], pkv)
            for k in range(chunk // 16):
                colv[pl.ds(k * 16, 16)] = pkv[0, pl.ds(k * 16, 16)]
                rowv[pl.ds(k * 16, 16)] = pkv[1, pl.ds(k * 16, 16)]
            pltpu.async_copy(nf_hbm.at[colv], rowsv, sem).wait()

            def scale_body(g, c2):
                ev16 = lax.bitcast_convert_type(pkv[2, pl.ds(g * 16, 16)], jnp.float32)
                for lane in range(16):
                    s = ev16[lane]
                    r = g * 16 + lane
                    for j in range(d // 16):
                        rowsv[r, pl.ds(j * 16, 16)] = rowsv[r, pl.ds(j * 16, 16)] * s
                return c2

            lax.fori_loop(0, chunk // 16, scale_body, 0)
            pltpu.sync_copy(rowsv, acc.at[rowv], add=True)
            return carry

        lax.fori_loop(0, ncp, chunk_body, 0)
        plsc.subcore_barrier()

        @pl.when(sid == 0)
        def _():
            pltpu.sync_copy(acc, out_hbm.at[cid])

    return spmm(nf, pk)


_ROWS = 2000


def _full(shape):
    return pl.BlockSpec(shape, lambda i: (0,) * len(shape))


def _rows(d):
    return pl.BlockSpec((_ROWS, d), lambda i: (i, 0))


def _mm(a, w, b):
    return jnp.dot(a, w, preferred_element_type=jnp.float32) + b


def _tc_lift_proj(nfp, w1p, b1, w2, b2, nw, nb, sw, sb):
    n, d = nfp.shape[0], w2.shape[0]

    def body(nf_ref, w1_ref, b1_ref, w2_ref, b2_ref, nw_ref, nb_ref,
             sw_ref, sb_ref, h_ref, nfo_ref, sfo_ref):
        a = _gelu(_mm(nf_ref[...], w1_ref[...], b1_ref[...]))
        h = _mm(a, w2_ref[...], b2_ref[...])
        h_ref[...] = h
        nfo_ref[...] = _mm(h, nw_ref[...], nb_ref[...])
        sfo_ref[...] = _mm(h, sw_ref[...], sb_ref[...])

    out = jax.ShapeDtypeStruct((n, d), jnp.float32)
    return pl.pallas_call(
        body,
        grid=(n // _ROWS,),
        in_specs=[_rows(d), _full((d, d)), _full((1, d)), _full((d, d)),
                  _full((1, d)), _full((d, d)), _full((1, d)),
                  _full((d, d)), _full((1, d))],
        out_specs=[_rows(d)] * 3,
        out_shape=[out, out, out],
    )(nfp, w1p, b1, w2, b2, nw, nb, sw, sb)


def _part_spec(which, d):
    return pl.BlockSpec((1, _ROWS, d), lambda i, w=which: (w, i, 0))


def _tc_gate_proj(h, sf, parts, g1wa, g1wb, g1b, g2w, g2b, nw, nb, sw, sb):
    n, d = h.shape

    def body(h_ref, sf_ref, p0_ref, p1_ref, g1wa_ref, g1wb_ref, g1b_ref,
             g2w_ref, g2b_ref, nw_ref, nb_ref, sw_ref, sb_ref,
             h_out, nfo_ref, sfo_ref):
        aggr = p0_ref[0] + p1_ref[0]
        t = _gelu(_mm(sf_ref[...], g1wa_ref[...], g1b_ref[...])
                  + jnp.dot(aggr, g1wb_ref[...], preferred_element_type=jnp.float32))
        hn = h_ref[...] + _mm(t, g2w_ref[...], g2b_ref[...])
        h_out[...] = hn
        nfo_ref[...] = _mm(hn, nw_ref[...], nb_ref[...])
        sfo_ref[...] = _mm(hn, sw_ref[...], sb_ref[...])

    out = jax.ShapeDtypeStruct((n, d), jnp.float32)
    return pl.pallas_call(
        body,
        grid=(n // _ROWS,),
        in_specs=[_rows(d), _rows(d), _part_spec(0, d), _part_spec(1, d),
                  _full((d, d)), _full((d, d)), _full((1, d)),
                  _full((d, d)), _full((1, d)), _full((d, d)), _full((1, d)),
                  _full((d, d)), _full((1, d))],
        out_specs=[_rows(d)] * 3,
        out_shape=[out, out, out],
    )(h, sf, parts, parts, g1wa, g1wb, g1b, g2w, g2b, nw, nb, sw, sb)


def _tc_gate_norm(h, sf, parts, g1wa, g1wb, g1b, g2w, g2b, gamma, beta):
    n, d = h.shape

    def body(h_ref, sf_ref, p0_ref, p1_ref, g1wa_ref, g1wb_ref, g1b_ref,
             g2w_ref, g2b_ref, gamma_ref, beta_ref, o_ref):
        aggr = p0_ref[0] + p1_ref[0]
        t = _gelu(_mm(sf_ref[...], g1wa_ref[...], g1b_ref[...])
                  + jnp.dot(aggr, g1wb_ref[...], preferred_element_type=jnp.float32))
        hn = h_ref[...] + _mm(t, g2w_ref[...], g2b_ref[...])
        mean = jnp.mean(hn, axis=-1, keepdims=True)
        cent = hn - mean
        var = jnp.mean(cent * cent, axis=-1, keepdims=True)
        o_ref[...] = cent * lax.rsqrt(var + 1e-5) * gamma_ref[...] + beta_ref[...]

    return pl.pallas_call(
        body,
        grid=(n // _ROWS,),
        in_specs=[_rows(d), _rows(d), _part_spec(0, d), _part_spec(1, d),
                  _full((d, d)), _full((d, d)), _full((1, d)),
                  _full((d, d)), _full((1, d)), _full((1, d)), _full((1, d))],
        out_specs=_rows(d),
        out_shape=jax.ShapeDtypeStruct((n, d), jnp.float32),
    )(h, sf, parts, parts, g1wa, g1wb, g1b, g2w, g2b, gamma, beta)


def kernel(x, edge_index, edge_values, lift_W1, lift_b1, lift_W2, lift_b2,
           sW0, sb0, nW0, nb0, g1W0, g1b0, g2W0, g2b0,
           sW1, sb1, nW1, nb1, g1W1, g1b1, g2W1, g2b1,
           gamma, beta):
    n = x.shape[1]
    d = lift_W1.shape[1]
    fin = lift_W1.shape[0]

    nfp = jnp.zeros((n, d), jnp.float32).at[:, :fin].set(x[0, :, 3:])
    w1p = jnp.zeros((d, d), jnp.float32).at[:fin, :].set(lift_W1)
    row = edge_index[0]
    col = edge_index[1]

    def r1(v):
        return v.reshape(1, d)

    h, nf0, sf0 = _tc_lift_proj(nfp, w1p, r1(lift_b1), lift_W2, r1(lift_b2),
                                nW0, r1(nb0), sW0, r1(sb0))
    parts0 = _sc_spmm(nf0, row, col, edge_values)
    h1, nf1, sf1 = _tc_gate_proj(h, sf0, parts0, g1W0[:d], g1W0[d:], r1(g1b0),
                                 g2W0, r1(g2b0), nW1, r1(nb1), sW1, r1(sb1))
    parts1 = _sc_spmm(nf1, row, col, edge_values)
    out = _tc_gate_norm(h1, sf1, parts1, g1W1[:d], g1W1[d:], r1(g1b1),
                        g2W1, r1(g2b1), r1(gamma), r1(beta))
    return out[None]


# speedup vs baseline: 1.0001x; 1.0001x over previous
import functools

import jax
import jax.numpy as jnp
from jax import lax
from jax.experimental import pallas as pl
from jax.experimental.pallas import tpu as pltpu
from jax.experimental.pallas import tpu_sc as plsc


_SQRT_HALF = 0.7071067811865476


def _gelu(v):
    return 0.5 * v * (1.0 + lax.erf(v * _SQRT_HALF))


def _pack_edges(row, col, ev):
    e = row.shape[0]
    nw, chunk = 32, 128
    ept = e // nw
    ncp = -(-ept // chunk)
    eptp = ncp * chunk

    def ptile(v):
        return jnp.pad(v.reshape(nw, ept), ((0, 0), (0, eptp - ept))
                       ).reshape(nw, ncp, chunk)

    pk = jnp.zeros((nw, ncp, 8, chunk), jnp.int32)
    pk = pk.at[:, :, 0].set(ptile(col))
    pk = pk.at[:, :, 1].set(ptile(row))
    pk = pk.at[:, :, 2].set(ptile(lax.bitcast_convert_type(ev, jnp.int32)))
    return pk


def _sc_spmm(nf, pk):
    n, d = nf.shape
    nc, ns = 2, 16
    nw = nc * ns
    chunk = 128
    ncp = pk.shape[1]
    rpt = n // ns
    zr = 25
    nz = rpt // zr

    mesh = plsc.VectorSubcoreMesh(core_axis_name="c", subcore_axis_name="s")

    @functools.partial(
        pl.kernel,
        mesh=mesh,
        out_type=jax.ShapeDtypeStruct((nc, n, d), jnp.float32),
        scratch_types=[
            pltpu.VMEM((8, chunk), jnp.int32),
            pltpu.VMEM((chunk,), jnp.int32),
            pltpu.VMEM((chunk,), jnp.int32),
            pltpu.VMEM((chunk, d), jnp.float32),
            pltpu.VMEM((zr, d), jnp.float32),
            pltpu.VMEM_SHARED((n, d), jnp.float32),
            pltpu.SemaphoreType.DMA,
        ],
    )
    def spmm(nf_hbm, pk_hbm, out_hbm,
             pkv, colv, rowv, rowsv, zbuf, acc, sem):
        cid = lax.axis_index("c")
        sid = lax.axis_index("s")
        tid = cid * ns + sid

        z16 = jnp.zeros((16,), jnp.float32)
        for i in range(zr):
            for j in range(d // 16):
                zbuf[i, pl.ds(j * 16, 16)] = z16
        zbase = sid * rpt

        def zero_body(i, carry):
            pltpu.sync_copy(zbuf, acc.at[pl.ds(zbase + i * zr, zr)])
            return carry

        lax.fori_loop(0, nz, zero_body, 0)
        plsc.subcore_barrier()

        def chunk_body(i, carry):
            pltpu.sync_copy(pk_hbm.at[tid, i], pkv)
            for k in range(chunk // 16):
                colv[pl.ds(k * 16, 16)] = pkv[0, pl.ds(k * 16, 16)]
                rowv[pl.ds(k * 16, 16)] = pkv[1, pl.ds(k * 16, 16)]
            pltpu.async_copy(nf_hbm.at[colv], rowsv, sem).wait()

            def scale_body(g, c2):
                ev16 = lax.bitcast_convert_type(pkv[2, pl.ds(g * 16, 16)], jnp.float32)
                for lane in range(16):
                    s = ev16[lane]
                    r = g * 16 + lane
                    for j in range(d // 16):
                        rowsv[r, pl.ds(j * 16, 16)] = rowsv[r, pl.ds(j * 16, 16)] * s
                return c2

            lax.fori_loop(0, chunk // 16, scale_body, 0)
            pltpu.sync_copy(rowsv, acc.at[rowv], add=True)
            return carry

        lax.fori_loop(0, ncp, chunk_body, 0)
        plsc.subcore_barrier()

        @pl.when(sid == 0)
        def _():
            pltpu.sync_copy(acc, out_hbm.at[cid])

    return spmm(nf, pk)


_ROWS = 2000


def _full(shape):
    return pl.BlockSpec(shape, lambda i: (0,) * len(shape))


def _rows(d):
    return pl.BlockSpec((_ROWS, d), lambda i: (i, 0))


def _mm(a, w, b):
    return jnp.dot(a, w, preferred_element_type=jnp.float32) + b


def _tc_lift_proj(nfp, w1p, b1, w2, b2, nw, nb, sw, sb):
    n, d = nfp.shape[0], w2.shape[0]

    def body(nf_ref, w1_ref, b1_ref, w2_ref, b2_ref, nw_ref, nb_ref,
             sw_ref, sb_ref, h_ref, nfo_ref, sfo_ref):
        a = _gelu(_mm(nf_ref[...], w1_ref[...], b1_ref[...]))
        h = _mm(a, w2_ref[...], b2_ref[...])
        h_ref[...] = h
        nfo_ref[...] = _mm(h, nw_ref[...], nb_ref[...])
        sfo_ref[...] = _mm(h, sw_ref[...], sb_ref[...])

    out = jax.ShapeDtypeStruct((n, d), jnp.float32)
    return pl.pallas_call(
        body,
        grid=(n // _ROWS,),
        in_specs=[_rows(d), _full((d, d)), _full((1, d)), _full((d, d)),
                  _full((1, d)), _full((d, d)), _full((1, d)),
                  _full((d, d)), _full((1, d))],
        out_specs=[_rows(d)] * 3,
        out_shape=[out, out, out],
    )(nfp, w1p, b1, w2, b2, nw, nb, sw, sb)


def _part_spec(which, d):
    return pl.BlockSpec((1, _ROWS, d), lambda i, w=which: (w, i, 0))


def _tc_gate_proj(h, sf, parts, g1wa, g1wb, g1b, g2w, g2b, nw, nb, sw, sb):
    n, d = h.shape

    def body(h_ref, sf_ref, p0_ref, p1_ref, g1wa_ref, g1wb_ref, g1b_ref,
             g2w_ref, g2b_ref, nw_ref, nb_ref, sw_ref, sb_ref,
             h_out, nfo_ref, sfo_ref):
        aggr = p0_ref[0] + p1_ref[0]
        t = _gelu(_mm(sf_ref[...], g1wa_ref[...], g1b_ref[...])
                  + jnp.dot(aggr, g1wb_ref[...], preferred_element_type=jnp.float32))
        hn = h_ref[...] + _mm(t, g2w_ref[...], g2b_ref[...])
        h_out[...] = hn
        nfo_ref[...] = _mm(hn, nw_ref[...], nb_ref[...])
        sfo_ref[...] = _mm(hn, sw_ref[...], sb_ref[...])

    out = jax.ShapeDtypeStruct((n, d), jnp.float32)
    return pl.pallas_call(
        body,
        grid=(n // _ROWS,),
        in_specs=[_rows(d), _rows(d), _part_spec(0, d), _part_spec(1, d),
                  _full((d, d)), _full((d, d)), _full((1, d)),
                  _full((d, d)), _full((1, d)), _full((d, d)), _full((1, d)),
                  _full((d, d)), _full((1, d))],
        out_specs=[_rows(d)] * 3,
        out_shape=[out, out, out],
    )(h, sf, parts, parts, g1wa, g1wb, g1b, g2w, g2b, nw, nb, sw, sb)


def _tc_gate_norm(h, sf, parts, g1wa, g1wb, g1b, g2w, g2b, gamma, beta):
    n, d = h.shape

    def body(h_ref, sf_ref, p0_ref, p1_ref, g1wa_ref, g1wb_ref, g1b_ref,
             g2w_ref, g2b_ref, gamma_ref, beta_ref, o_ref):
        aggr = p0_ref[0] + p1_ref[0]
        t = _gelu(_mm(sf_ref[...], g1wa_ref[...], g1b_ref[...])
                  + jnp.dot(aggr, g1wb_ref[...], preferred_element_type=jnp.float32))
        hn = h_ref[...] + _mm(t, g2w_ref[...], g2b_ref[...])
        mean = jnp.mean(hn, axis=-1, keepdims=True)
        cent = hn - mean
        var = jnp.mean(cent * cent, axis=-1, keepdims=True)
        o_ref[...] = cent * lax.rsqrt(var + 1e-5) * gamma_ref[...] + beta_ref[...]

    return pl.pallas_call(
        body,
        grid=(n // _ROWS,),
        in_specs=[_rows(d), _rows(d), _part_spec(0, d), _part_spec(1, d),
                  _full((d, d)), _full((d, d)), _full((1, d)),
                  _full((d, d)), _full((1, d)), _full((1, d)), _full((1, d))],
        out_specs=_rows(d),
        out_shape=jax.ShapeDtypeStruct((n, d), jnp.float32),
    )(h, sf, parts, parts, g1wa, g1wb, g1b, g2w, g2b, gamma, beta)


def kernel(x, edge_index, edge_values, lift_W1, lift_b1, lift_W2, lift_b2,
           sW0, sb0, nW0, nb0, g1W0, g1b0, g2W0, g2b0,
           sW1, sb1, nW1, nb1, g1W1, g1b1, g2W1, g2b1,
           gamma, beta):
    n = x.shape[1]
    d = lift_W1.shape[1]
    fin = lift_W1.shape[0]

    nfp = jnp.zeros((n, d), jnp.float32).at[:, :fin].set(x[0, :, 3:])
    w1p = jnp.zeros((d, d), jnp.float32).at[:fin, :].set(lift_W1)
    pk = _pack_edges(edge_index[0], edge_index[1], edge_values)

    def r1(v):
        return v.reshape(1, d)

    h, nf0, sf0 = _tc_lift_proj(nfp, w1p, r1(lift_b1), lift_W2, r1(lift_b2),
                                nW0, r1(nb0), sW0, r1(sb0))
    parts0 = _sc_spmm(nf0, pk)
    h1, nf1, sf1 = _tc_gate_proj(h, sf0, parts0, g1W0[:d], g1W0[d:], r1(g1b0),
                                 g2W0, r1(g2b0), nW1, r1(nb1), sW1, r1(sb1))
    parts1 = _sc_spmm(nf1, pk)
    out = _tc_gate_norm(h1, sf1, parts1, g1W1[:d], g1W1[d:], r1(g1b1),
                        g2W1, r1(g2b1), r1(gamma), r1(beta))
    return out[None]


# speedup vs baseline: 1.6564x; 1.6563x over previous
import functools

import jax
import jax.numpy as jnp
from jax import lax
from jax.experimental import pallas as pl
from jax.experimental.pallas import tpu as pltpu
from jax.experimental.pallas import tpu_sc as plsc


_SQRT_HALF = 0.7071067811865476


def _gelu(v):
    return 0.5 * v * (1.0 + lax.erf(v * _SQRT_HALF))


def _sc_spmm(nf, row, col, ev):
    n, d = nf.shape
    e = row.shape[0]
    nc, ns = 2, 16
    nw = nc * ns
    chunk = 128
    ept = e // nw
    nchunks = ept // chunk
    rem = ept - nchunks * chunk
    rpt = n // ns
    zr = 25
    nz = rpt // zr

    mesh = plsc.VectorSubcoreMesh(core_axis_name="c", subcore_axis_name="s")

    @functools.partial(
        pl.kernel,
        mesh=mesh,
        out_type=jax.ShapeDtypeStruct((nc, n, d), jnp.float32),
        scratch_types=[
            pltpu.VMEM((chunk,), jnp.int32),
            pltpu.VMEM((chunk,), jnp.int32),
            pltpu.VMEM((chunk,), jnp.float32),
            pltpu.VMEM((chunk, d), jnp.float32),
            pltpu.VMEM((max(rem, 16),), jnp.int32),
            pltpu.VMEM((max(rem, 16),), jnp.int32),
            pltpu.VMEM((max(rem, 16),), jnp.float32),
            pltpu.VMEM((max(rem, 16), d), jnp.float32),
            pltpu.VMEM((zr, d), jnp.float32),
            pltpu.VMEM_SHARED((n, d), jnp.float32),
            pltpu.SemaphoreType.DMA,
        ],
    )
    def spmm(nf_hbm, row_hbm, col_hbm, ev_hbm, out_hbm,
             colv, rowv, evv, rowsv, colr, rowr, evr, rowsr, zbuf, acc, sem):
        cid = lax.axis_index("c")
        sid = lax.axis_index("s")

        z16 = jnp.zeros((16,), jnp.float32)
        for i in range(zr):
            for j in range(d // 16):
                zbuf[i, pl.ds(j * 16, 16)] = z16
        zbase = sid * rpt

        zcs = [pltpu.async_copy(zbuf, acc.at[pl.ds(zbase + i * zr, zr)], sem)
               for i in range(nz)]
        for zc in zcs:
            zc.wait()
        plsc.subcore_barrier()

        base = (cid * ns + sid) * ept

        def process(off, cw, colb, rowb, evb, rowsb):
            c1 = pltpu.async_copy(col_hbm.at[pl.ds(off, cw)], colb, sem)
            c2 = pltpu.async_copy(row_hbm.at[pl.ds(off, cw)], rowb, sem)
            c3 = pltpu.async_copy(ev_hbm.at[pl.ds(off, cw)], evb, sem)
            c1.wait()
            c2.wait()
            c3.wait()
            pltpu.async_copy(nf_hbm.at[colb], rowsb, sem).wait()

            def scale_body(g, c2):
                ev16 = evb[pl.ds(g * 16, 16)]
                for lane in range(16):
                    s = ev16[lane]
                    r = g * 16 + lane
                    for j in range(d // 16):
                        rowsb[r, pl.ds(j * 16, 16)] = rowsb[r, pl.ds(j * 16, 16)] * s
                return c2

            lax.fori_loop(0, cw // 16, scale_body, 0)
            pltpu.sync_copy(rowsb, acc.at[rowb], add=True)

        def chunk_body(i, carry):
            process(base + i * chunk, chunk, colv, rowv, evv, rowsv)
            return carry

        lax.fori_loop(0, nchunks, chunk_body, 0)
        if rem:
            process(base + nchunks * chunk, rem, colr, rowr, evr, rowsr)
        plsc.subcore_barrier()

        @pl.when(sid == 0)
        def _():
            pltpu.sync_copy(acc, out_hbm.at[cid])

    return spmm(nf, row, col, ev)


_ROWS = 2000


def _full(shape):
    return pl.BlockSpec(shape, lambda i: (0,) * len(shape))


def _rows(d):
    return pl.BlockSpec((_ROWS, d), lambda i: (i, 0))


def _mm(a, w, b):
    return jnp.dot(a, w, preferred_element_type=jnp.float32) + b


def _tc_lift_proj(nfp, w1p, b1, w2, b2, nw, nb, sw, sb):
    n, d = nfp.shape[0], w2.shape[0]

    def body(nf_ref, w1_ref, b1_ref, w2_ref, b2_ref, nw_ref, nb_ref,
             sw_ref, sb_ref, h_ref, nfo_ref, sfo_ref):
        a = _gelu(_mm(nf_ref[...], w1_ref[...], b1_ref[...]))
        h = _mm(a, w2_ref[...], b2_ref[...])
        h_ref[...] = h
        nfo_ref[...] = _mm(h, nw_ref[...], nb_ref[...])
        sfo_ref[...] = _mm(h, sw_ref[...], sb_ref[...])

    out = jax.ShapeDtypeStruct((n, d), jnp.float32)
    return pl.pallas_call(
        body,
        grid=(n // _ROWS,),
        in_specs=[_rows(d), _full((d, d)), _full((1, d)), _full((d, d)),
                  _full((1, d)), _full((d, d)), _full((1, d)),
                  _full((d, d)), _full((1, d))],
        out_specs=[_rows(d)] * 3,
        out_shape=[out, out, out],
    )(nfp, w1p, b1, w2, b2, nw, nb, sw, sb)


def _part_spec(which, d):
    return pl.BlockSpec((1, _ROWS, d), lambda i, w=which: (w, i, 0))


def _tc_gate_proj(h, sf, parts, g1wa, g1wb, g1b, g2w, g2b, nw, nb, sw, sb):
    n, d = h.shape

    def body(h_ref, sf_ref, p0_ref, p1_ref, g1wa_ref, g1wb_ref, g1b_ref,
             g2w_ref, g2b_ref, nw_ref, nb_ref, sw_ref, sb_ref,
             h_out, nfo_ref, sfo_ref):
        aggr = p0_ref[0] + p1_ref[0]
        t = _gelu(_mm(sf_ref[...], g1wa_ref[...], g1b_ref[...])
                  + jnp.dot(aggr, g1wb_ref[...], preferred_element_type=jnp.float32))
        hn = h_ref[...] + _mm(t, g2w_ref[...], g2b_ref[...])
        h_out[...] = hn
        nfo_ref[...] = _mm(hn, nw_ref[...], nb_ref[...])
        sfo_ref[...] = _mm(hn, sw_ref[...], sb_ref[...])

    out = jax.ShapeDtypeStruct((n, d), jnp.float32)
    return pl.pallas_call(
        body,
        grid=(n // _ROWS,),
        in_specs=[_rows(d), _rows(d), _part_spec(0, d), _part_spec(1, d),
                  _full((d, d)), _full((d, d)), _full((1, d)),
                  _full((d, d)), _full((1, d)), _full((d, d)), _full((1, d)),
                  _full((d, d)), _full((1, d))],
        out_specs=[_rows(d)] * 3,
        out_shape=[out, out, out],
    )(h, sf, parts, parts, g1wa, g1wb, g1b, g2w, g2b, nw, nb, sw, sb)


def _tc_gate_norm(h, sf, parts, g1wa, g1wb, g1b, g2w, g2b, gamma, beta):
    n, d = h.shape

    def body(h_ref, sf_ref, p0_ref, p1_ref, g1wa_ref, g1wb_ref, g1b_ref,
             g2w_ref, g2b_ref, gamma_ref, beta_ref, o_ref):
        aggr = p0_ref[0] + p1_ref[0]
        t = _gelu(_mm(sf_ref[...], g1wa_ref[...], g1b_ref[...])
                  + jnp.dot(aggr, g1wb_ref[...], preferred_element_type=jnp.float32))
        hn = h_ref[...] + _mm(t, g2w_ref[...], g2b_ref[...])
        mean = jnp.mean(hn, axis=-1, keepdims=True)
        cent = hn - mean
        var = jnp.mean(cent * cent, axis=-1, keepdims=True)
        o_ref[...] = cent * lax.rsqrt(var + 1e-5) * gamma_ref[...] + beta_ref[...]

    return pl.pallas_call(
        body,
        grid=(n // _ROWS,),
        in_specs=[_rows(d), _rows(d), _part_spec(0, d), _part_spec(1, d),
                  _full((d, d)), _full((d, d)), _full((1, d)),
                  _full((d, d)), _full((1, d)), _full((1, d)), _full((1, d))],
        out_specs=_rows(d),
        out_shape=jax.ShapeDtypeStruct((n, d), jnp.float32),
    )(h, sf, parts, parts, g1wa, g1wb, g1b, g2w, g2b, gamma, beta)


def kernel(x, edge_index, edge_values, lift_W1, lift_b1, lift_W2, lift_b2,
           sW0, sb0, nW0, nb0, g1W0, g1b0, g2W0, g2b0,
           sW1, sb1, nW1, nb1, g1W1, g1b1, g2W1, g2b1,
           gamma, beta):
    n = x.shape[1]
    d = lift_W1.shape[1]
    fin = lift_W1.shape[0]

    nfp = jnp.zeros((n, d), jnp.float32).at[:, :fin].set(x[0, :, 3:])
    w1p = jnp.zeros((d, d), jnp.float32).at[:fin, :].set(lift_W1)
    row = edge_index[0]
    col = edge_index[1]

    def r1(v):
        return v.reshape(1, d)

    h, nf0, sf0 = _tc_lift_proj(nfp, w1p, r1(lift_b1), lift_W2, r1(lift_b2),
                                nW0, r1(nb0), sW0, r1(sb0))
    parts0 = _sc_spmm(nf0, row, col, edge_values)
    h1, nf1, sf1 = _tc_gate_proj(h, sf0, parts0, g1W0[:d], g1W0[d:], r1(g1b0),
                                 g2W0, r1(g2b0), nW1, r1(nb1), sW1, r1(sb1))
    parts1 = _sc_spmm(nf1, row, col, edge_values)
    out = _tc_gate_norm(h1, sf1, parts1, g1W1[:d], g1W1[d:], r1(g1b1),
                        g2W1, r1(g2b1), r1(gamma), r1(beta))
    return out[None]


# speedup vs baseline: 1.9630x; 1.1851x over previous
import functools

import jax
import jax.numpy as jnp
from jax import lax
from jax.experimental import pallas as pl
from jax.experimental.pallas import tpu as pltpu
from jax.experimental.pallas import tpu_sc as plsc


_SQRT_HALF = 0.7071067811865476


def _gelu(v):
    return 0.5 * v * (1.0 + lax.erf(v * _SQRT_HALF))


def _sc_spmm(nf, row, col, ev):
    n, d = nf.shape
    e = row.shape[0]
    nc, ns = 2, 16
    nw = nc * ns
    chunk = 128
    ept = e // nw
    nchunks = ept // chunk
    rem = ept - nchunks * chunk
    rpt = n // ns
    zr = 25
    nz = rpt // zr

    mesh = plsc.VectorSubcoreMesh(core_axis_name="c", subcore_axis_name="s")

    @functools.partial(
        pl.kernel,
        mesh=mesh,
        out_type=jax.ShapeDtypeStruct((nc, n, d), jnp.float32),
        scratch_types=[
            pltpu.VMEM((chunk,), jnp.int32),
            pltpu.VMEM((chunk,), jnp.int32),
            pltpu.VMEM((chunk,), jnp.float32),
            pltpu.VMEM((chunk, d), jnp.float32),
            pltpu.VMEM((chunk,), jnp.int32),
            pltpu.VMEM((chunk,), jnp.int32),
            pltpu.VMEM((chunk,), jnp.float32),
            pltpu.VMEM((chunk, d), jnp.float32),
            pltpu.VMEM((max(rem, 16),), jnp.int32),
            pltpu.VMEM((max(rem, 16),), jnp.int32),
            pltpu.VMEM((max(rem, 16),), jnp.float32),
            pltpu.VMEM((max(rem, 16), d), jnp.float32),
            pltpu.VMEM((zr, d), jnp.float32),
            pltpu.VMEM_SHARED((n, d), jnp.float32),
            pltpu.SemaphoreType.DMA,
            pltpu.SemaphoreType.DMA,
        ],
    )
    def spmm(nf_hbm, row_hbm, col_hbm, ev_hbm, out_hbm,
             colv, rowv, evv, rowsv, colv2, rowv2, evv2, rowsv2,
             colr, rowr, evr, rowsr, zbuf, acc, sem, sem2):
        cid = lax.axis_index("c")
        sid = lax.axis_index("s")

        z16 = jnp.zeros((16,), jnp.float32)
        for i in range(zr):
            for j in range(d // 16):
                zbuf[i, pl.ds(j * 16, 16)] = z16
        zbase = sid * rpt

        zcs = [pltpu.async_copy(zbuf, acc.at[pl.ds(zbase + i * zr, zr)], sem)
               for i in range(nz)]
        for zc in zcs:
            zc.wait()
        plsc.subcore_barrier()

        base = (cid * ns + sid) * ept

        def load_idx(off, cw, colb, rowb, evb, s_):
            return (pltpu.async_copy(col_hbm.at[pl.ds(off, cw)], colb, s_),
                    pltpu.async_copy(row_hbm.at[pl.ds(off, cw)], rowb, s_),
                    pltpu.async_copy(ev_hbm.at[pl.ds(off, cw)], evb, s_))

        def scale_scatter(cw, rowb, evb, rowsb):
            def scale_body(g, c2):
                ev16 = evb[pl.ds(g * 16, 16)]
                for lane in range(16):
                    s = ev16[lane]
                    r = g * 16 + lane
                    for j in range(d // 16):
                        rowsb[r, pl.ds(j * 16, 16)] = rowsb[r, pl.ds(j * 16, 16)] * s
                return c2

            lax.fori_loop(0, cw // 16, scale_body, 0)
            pltpu.sync_copy(rowsb, acc.at[rowb], add=True)

        def process(off, cw, colb, rowb, evb, rowsb):
            for c in load_idx(off, cw, colb, rowb, evb, sem):
                c.wait()
            pltpu.async_copy(nf_hbm.at[colb], rowsb, sem).wait()
            scale_scatter(cw, rowb, evb, rowsb)

        def pair_body(i, carry):
            offa = base + (2 * i) * chunk
            offb = offa + chunk
            ca = load_idx(offa, chunk, colv, rowv, evv, sem)
            cb = load_idx(offb, chunk, colv2, rowv2, evv2, sem2)
            for c in ca:
                c.wait()
            ga = pltpu.async_copy(nf_hbm.at[colv], rowsv, sem)
            for c in cb:
                c.wait()
            gb = pltpu.async_copy(nf_hbm.at[colv2], rowsv2, sem2)
            ga.wait()
            scale_scatter(chunk, rowv, evv, rowsv)
            gb.wait()
            scale_scatter(chunk, rowv2, evv2, rowsv2)
            return carry

        npairs = nchunks // 2
        lax.fori_loop(0, npairs, pair_body, 0)
        for i in range(npairs * 2, nchunks):
            process(base + i * chunk, chunk, colv, rowv, evv, rowsv)
        if rem:
            process(base + nchunks * chunk, rem, colr, rowr, evr, rowsr)
        plsc.subcore_barrier()

        @pl.when(sid == 0)
        def _():
            pltpu.sync_copy(acc, out_hbm.at[cid])

    return spmm(nf, row, col, ev)


_ROWS = 2000


def _full(shape):
    return pl.BlockSpec(shape, lambda i: (0,) * len(shape))


def _rows(d):
    return pl.BlockSpec((_ROWS, d), lambda i: (i, 0))


def _mm(a, w, b):
    return jnp.dot(a, w, preferred_element_type=jnp.float32) + b


def _tc_lift_proj(nfp, w1p, b1, w2, b2, nw, nb, sw, sb):
    n, d = nfp.shape[0], w2.shape[0]

    def body(nf_ref, w1_ref, b1_ref, w2_ref, b2_ref, nw_ref, nb_ref,
             sw_ref, sb_ref, h_ref, nfo_ref, sfo_ref):
        a = _gelu(_mm(nf_ref[...], w1_ref[...], b1_ref[...]))
        h = _mm(a, w2_ref[...], b2_ref[...])
        h_ref[...] = h
        nfo_ref[...] = _mm(h, nw_ref[...], nb_ref[...])
        sfo_ref[...] = _mm(h, sw_ref[...], sb_ref[...])

    out = jax.ShapeDtypeStruct((n, d), jnp.float32)
    return pl.pallas_call(
        body,
        grid=(n // _ROWS,),
        in_specs=[_rows(d), _full((d, d)), _full((1, d)), _full((d, d)),
                  _full((1, d)), _full((d, d)), _full((1, d)),
                  _full((d, d)), _full((1, d))],
        out_specs=[_rows(d)] * 3,
        out_shape=[out, out, out],
    )(nfp, w1p, b1, w2, b2, nw, nb, sw, sb)


def _part_spec(which, d):
    return pl.BlockSpec((1, _ROWS, d), lambda i, w=which: (w, i, 0))


def _tc_gate_proj(h, sf, parts, g1wa, g1wb, g1b, g2w, g2b, nw, nb, sw, sb):
    n, d = h.shape

    def body(h_ref, sf_ref, p0_ref, p1_ref, g1wa_ref, g1wb_ref, g1b_ref,
             g2w_ref, g2b_ref, nw_ref, nb_ref, sw_ref, sb_ref,
             h_out, nfo_ref, sfo_ref):
        aggr = p0_ref[0] + p1_ref[0]
        t = _gelu(_mm(sf_ref[...], g1wa_ref[...], g1b_ref[...])
                  + jnp.dot(aggr, g1wb_ref[...], preferred_element_type=jnp.float32))
        hn = h_ref[...] + _mm(t, g2w_ref[...], g2b_ref[...])
        h_out[...] = hn
        nfo_ref[...] = _mm(hn, nw_ref[...], nb_ref[...])
        sfo_ref[...] = _mm(hn, sw_ref[...], sb_ref[...])

    out = jax.ShapeDtypeStruct((n, d), jnp.float32)
    return pl.pallas_call(
        body,
        grid=(n // _ROWS,),
        in_specs=[_rows(d), _rows(d), _part_spec(0, d), _part_spec(1, d),
                  _full((d, d)), _full((d, d)), _full((1, d)),
                  _full((d, d)), _full((1, d)), _full((d, d)), _full((1, d)),
                  _full((d, d)), _full((1, d))],
        out_specs=[_rows(d)] * 3,
        out_shape=[out, out, out],
    )(h, sf, parts, parts, g1wa, g1wb, g1b, g2w, g2b, nw, nb, sw, sb)


def _tc_gate_norm(h, sf, parts, g1wa, g1wb, g1b, g2w, g2b, gamma, beta):
    n, d = h.shape

    def body(h_ref, sf_ref, p0_ref, p1_ref, g1wa_ref, g1wb_ref, g1b_ref,
             g2w_ref, g2b_ref, gamma_ref, beta_ref, o_ref):
        aggr = p0_ref[0] + p1_ref[0]
        t = _gelu(_mm(sf_ref[...], g1wa_ref[...], g1b_ref[...])
                  + jnp.dot(aggr, g1wb_ref[...], preferred_element_type=jnp.float32))
        hn = h_ref[...] + _mm(t, g2w_ref[...], g2b_ref[...])
        mean = jnp.mean(hn, axis=-1, keepdims=True)
        cent = hn - mean
        var = jnp.mean(cent * cent, axis=-1, keepdims=True)
        o_ref[...] = cent * lax.rsqrt(var + 1e-5) * gamma_ref[...] + beta_ref[...]

    return pl.pallas_call(
        body,
        grid=(n // _ROWS,),
        in_specs=[_rows(d), _rows(d), _part_spec(0, d), _part_spec(1, d),
                  _full((d, d)), _full((d, d)), _full((1, d)),
                  _full((d, d)), _full((1, d)), _full((1, d)), _full((1, d))],
        out_specs=_rows(d),
        out_shape=jax.ShapeDtypeStruct((n, d), jnp.float32),
    )(h, sf, parts, parts, g1wa, g1wb, g1b, g2w, g2b, gamma, beta)


def kernel(x, edge_index, edge_values, lift_W1, lift_b1, lift_W2, lift_b2,
           sW0, sb0, nW0, nb0, g1W0, g1b0, g2W0, g2b0,
           sW1, sb1, nW1, nb1, g1W1, g1b1, g2W1, g2b1,
           gamma, beta):
    n = x.shape[1]
    d = lift_W1.shape[1]
    fin = lift_W1.shape[0]

    nfp = jnp.zeros((n, d), jnp.float32).at[:, :fin].set(x[0, :, 3:])
    w1p = jnp.zeros((d, d), jnp.float32).at[:fin, :].set(lift_W1)
    row = edge_index[0]
    col = edge_index[1]

    def r1(v):
        return v.reshape(1, d)

    h, nf0, sf0 = _tc_lift_proj(nfp, w1p, r1(lift_b1), lift_W2, r1(lift_b2),
                                nW0, r1(nb0), sW0, r1(sb0))
    parts0 = _sc_spmm(nf0, row, col, edge_values)
    h1, nf1, sf1 = _tc_gate_proj(h, sf0, parts0, g1W0[:d], g1W0[d:], r1(g1b0),
                                 g2W0, r1(g2b0), nW1, r1(nb1), sW1, r1(sb1))
    parts1 = _sc_spmm(nf1, row, col, edge_values)
    out = _tc_gate_norm(h1, sf1, parts1, g1W1[:d], g1W1[d:], r1(g1b1),
                        g2W1, r1(g2b1), r1(gamma), r1(beta))
    return out[None]
